# baseline (device time: 76046 ns/iter reference)
import functools

import jax
import jax.numpy as jnp
from jax import lax
from jax.experimental import pallas as pl
from jax.experimental.pallas import tpu as pltpu

N_DEV = 4
SQ = 256
SH = 128
D_MODEL = 1024
HG = 8
DH = 128
SKV = 4096
SCALE = 0.08838834764831843
NEG = -1e9
W = 384
G = 128


def kernel(x, Wq, K_ext, V_ext, Wo):
    x_bf = x[0].astype(jnp.bfloat16)
    wq_bf = Wq.astype(jnp.bfloat16)
    wo_bf = Wo.astype(jnp.bfloat16)

    def body(x_ref, wq_ref, k_hbm, v_hbm, wo_ref, out_ref,
             k_ref, v_ref, kvstage,
             xbT, xbB, pT, pB, rsT_s, rsT_r, rsB_s, rsB_r,
             dsem, agT_ss, agT_rs, agB_ss, agB_rs,
             rsT_ss, rsT_rs, rsB_ss, rsB_rs):
        my_i = lax.axis_index("i")
        left = (my_i + N_DEV - 1) % N_DEV
        right = (my_i + 1) % N_DEV

        def stage_head(h):
            buf = h % 2
            ck = pltpu.make_async_copy(
                k_hbm.at[0, :, my_i * HG + h, :], kvstage.at[buf, 0],
                dsem.at[buf, 0])
            cv = pltpu.make_async_copy(
                v_hbm.at[0, :, my_i * HG + h, :], kvstage.at[buf, 1],
                dsem.at[buf, 1])
            ck.start()
            cv.start()
            return ck, cv

        pend = {0: stage_head(0), 1: stage_head(1)}

        bsem = pltpu.get_barrier_semaphore()
        for nbr in (left, right):
            pl.semaphore_signal(bsem, inc=1, device_id=(nbr,),
                                device_id_type=pl.DeviceIdType.MESH)
        pl.semaphore_wait(bsem, 2)

        def ag_T(h):
            src = x_ref.at[pl.ds(0, SH)] if h == 0 else xbT.at[h - 1]
            return pltpu.make_async_remote_copy(
                src_ref=src, dst_ref=xbT.at[h],
                send_sem=agT_ss.at[h], recv_sem=agT_rs.at[h],
                device_id=(right,), device_id_type=pl.DeviceIdType.MESH)

        def ag_B(h):
            src = x_ref.at[pl.ds(SH, SH)] if h == 0 else xbB.at[h - 1]
            return pltpu.make_async_remote_copy(
                src_ref=src, dst_ref=xbB.at[h],
                send_sem=agB_ss.at[h], recv_sem=agB_rs.at[h],
                device_id=(left,), device_id_type=pl.DeviceIdType.MESH)

        def rs_T(t):
            return pltpu.make_async_remote_copy(
                src_ref=rsT_s.at[t], dst_ref=rsT_r.at[t],
                send_sem=rsT_ss.at[t], recv_sem=rsT_rs.at[t],
                device_id=(right,), device_id_type=pl.DeviceIdType.MESH)

        def rs_B(t):
            return pltpu.make_async_remote_copy(
                src_ref=rsB_s.at[t], dst_ref=rsB_r.at[t],
                send_sem=rsB_ss.at[t], recv_sem=rsB_rs.at[t],
                device_id=(left,), device_id_type=pl.DeviceIdType.MESH)

        def softmax_rows(s, mask):
            s = jnp.where(mask, s, jnp.float32(NEG)) if mask is not None else s
            m = jnp.max(s, axis=1, keepdims=True)
            w = jnp.exp(s - m)
            d = jnp.sum(w, axis=1, keepdims=True)
            return (w / d).astype(jnp.bfloat16)

        def compute_half(dst, slot, q, top):
            if top:
                xq = x_ref[0:SH] if slot == 0 else xbT[slot - 1]
            else:
                xq = x_ref[SH:SQ] if slot == 0 else xbB[slot - 1]
            qm = jnp.dot(xq, wq_ref[:], preferred_element_type=jnp.float32)
            qm = (qm * SCALE).astype(jnp.bfloat16)
            base = q * SQ + (0 if top else SH)
            start = pl.multiple_of(jnp.maximum(base - SH, 0), 128)
            qi = base + lax.broadcasted_iota(jnp.int32, (SH, W), 0)
            kiw = start + lax.broadcasted_iota(jnp.int32, (SH, W), 1)
            mask_w = (jnp.abs(qi - kiw) <= 128) | (kiw < 32)
            kig = lax.broadcasted_iota(jnp.int32, (SH, G), 1)
            mask_g = (kig < 32) & (start > 0)
            mask = jnp.concatenate([mask_w, mask_g], axis=1)
            parts = []
            for h in range(HG):
                qh = qm[:, h * DH:(h + 1) * DH]
                kcat = jnp.concatenate(
                    [k_ref[h, pl.ds(start, W), :], k_ref[h, :G, :]], axis=0)
                vcat = jnp.concatenate(
                    [v_ref[h, pl.ds(start, W), :], v_ref[h, :G, :]], axis=0)
                s = lax.dot_general(qh, kcat, (((1,), (1,)), ((), ())),
                                    preferred_element_type=jnp.float32)
                w = softmax_rows(s, mask)
                parts.append(jnp.dot(w, vcat,
                                     preferred_element_type=jnp.float32))
            ctx = jnp.concatenate(parts, axis=1).astype(jnp.bfloat16)
            dst[slot] = jnp.dot(ctx, wo_ref[:],
                                preferred_element_type=jnp.float32)

            if top:
                @pl.when(q == 0)
                def _():
                    parts32 = []
                    for h in range(HG):
                        q32 = qm[0:32, h * DH:(h + 1) * DH]
                        s32 = lax.dot_general(
                            q32, k_ref[h], (((1,), (1,)), ((), ())),
                            preferred_element_type=jnp.float32)
                        w32 = softmax_rows(s32, None)
                        parts32.append(jnp.dot(
                            w32, v_ref[h], preferred_element_type=jnp.float32))
                    ctx32 = jnp.concatenate(
                        parts32, axis=1).astype(jnp.bfloat16)
                    dst[slot, 0:32, :] = jnp.dot(
                        ctx32, wo_ref[:], preferred_element_type=jnp.float32)

        agT0 = ag_T(0)
        agT0.start()
        agB0 = ag_B(0)
        agB0.start()

        for h in range(HG):
            ck, cv = pend[h]
            ck.wait()
            cv.wait()
            k_ref[h] = kvstage[h % 2, 0].astype(jnp.bfloat16)
            v_ref[h] = kvstage[h % 2, 1].astype(jnp.bfloat16)
            if h + 2 < HG:
                pend[h + 2] = stage_head(h + 2)

        compute_half(pT, 0, my_i, True)
        compute_half(pB, 0, my_i, False)

        agT0.wait()
        agT1 = ag_T(1)
        agT1.start()
        agB0.wait()
        agB1 = ag_B(1)
        agB1.start()

        compute_half(pT, 1, (my_i + 3) % N_DEV, True)
        rsT_s[0] = pT[1].astype(jnp.bfloat16)
        rsT0 = rs_T(0)
        rsT0.start()
        compute_half(pB, 1, (my_i + 1) % N_DEV, False)
        rsB_s[0] = pB[1].astype(jnp.bfloat16)
        rsB0 = rs_B(0)
        rsB0.start()

        agT1.wait()
        agT2 = ag_T(2)
        agT2.start()
        agB1.wait()
        agB2 = ag_B(2)
        agB2.start()

        compute_half(pT, 2, (my_i + 2) % N_DEV, True)
        rsT0.wait()
        pT[2] += rsT_r[0].astype(jnp.float32)
        rsT_s[1] = pT[2].astype(jnp.bfloat16)
        rsT1 = rs_T(1)
        rsT1.start()
        compute_half(pB, 2, (my_i + 2) % N_DEV, False)
        rsB0.wait()
        pB[2] += rsB_r[0].astype(jnp.float32)
        rsB_s[1] = pB[2].astype(jnp.bfloat16)
        rsB1 = rs_B(1)
        rsB1.start()

        agT2.wait()
        agB2.wait()

        compute_half(pT, 3, (my_i + 1) % N_DEV, True)
        rsT1.wait()
        pT[3] += rsT_r[1].astype(jnp.float32)
        rsT_s[2] = pT[3].astype(jnp.bfloat16)
        rsT2 = rs_T(2)
        rsT2.start()
        compute_half(pB, 3, (my_i + 3) % N_DEV, False)
        rsB1.wait()
        pB[3] += rsB_r[1].astype(jnp.float32)
        rsB_s[2] = pB[3].astype(jnp.bfloat16)
        rsB2 = rs_B(2)
        rsB2.start()

        rsT2.wait()
        out_ref[0:SH, :] = pT[0] + rsT_r[2].astype(jnp.float32)
        rsB2.wait()
        out_ref[SH:SQ, :] = pB[0] + rsB_r[2].astype(jnp.float32)

        @functools.partial(pl.run_scoped, sem2=pltpu.SemaphoreType.REGULAR)
        def _(sem2):
            for nbr in (left, right):
                pl.semaphore_signal(sem2, inc=1, device_id=(nbr,),
                                    device_id_type=pl.DeviceIdType.MESH)
            pl.semaphore_wait(sem2, 2)

    out = pl.pallas_call(
        body,
        out_shape=jax.ShapeDtypeStruct((SQ, D_MODEL), jnp.float32),
        in_specs=[
            pl.BlockSpec(memory_space=pltpu.VMEM),
            pl.BlockSpec(memory_space=pltpu.VMEM),
            pl.BlockSpec(memory_space=pl.ANY),
            pl.BlockSpec(memory_space=pl.ANY),
            pl.BlockSpec(memory_space=pltpu.VMEM),
        ],
        out_specs=pl.BlockSpec(memory_space=pltpu.VMEM),
        scratch_shapes=[
            pltpu.VMEM((HG, SKV, DH), jnp.bfloat16),
            pltpu.VMEM((HG, SKV, DH), jnp.bfloat16),
            pltpu.VMEM((2, 2, SKV, DH), jnp.float32),
            pltpu.VMEM((3, SH, D_MODEL), jnp.bfloat16),
            pltpu.VMEM((3, SH, D_MODEL), jnp.bfloat16),
            pltpu.VMEM((N_DEV, SH, D_MODEL), jnp.float32),
            pltpu.VMEM((N_DEV, SH, D_MODEL), jnp.float32),
            pltpu.VMEM((3, SH, D_MODEL), jnp.bfloat16),
            pltpu.VMEM((3, SH, D_MODEL), jnp.bfloat16),
            pltpu.VMEM((3, SH, D_MODEL), jnp.bfloat16),
            pltpu.VMEM((3, SH, D_MODEL), jnp.bfloat16),
            pltpu.SemaphoreType.DMA((2, 2)),
            pltpu.SemaphoreType.DMA((3,)),
            pltpu.SemaphoreType.DMA((3,)),
            pltpu.SemaphoreType.DMA((3,)),
            pltpu.SemaphoreType.DMA((3,)),
            pltpu.SemaphoreType.DMA((3,)),
            pltpu.SemaphoreType.DMA((3,)),
            pltpu.SemaphoreType.DMA((3,)),
            pltpu.SemaphoreType.DMA((3,)),
        ],
        compiler_params=pltpu.CompilerParams(
            collective_id=0, vmem_limit_bytes=60 * 1024 * 1024),
    )(x_bf, wq_bf, K_ext, V_ext, wo_bf)
    return out[None]


# device time: 75508 ns/iter; 1.0071x vs baseline; 1.0071x over previous
import functools

import jax
import jax.numpy as jnp
from jax import lax
from jax.experimental import pallas as pl
from jax.experimental.pallas import tpu as pltpu

N_DEV = 4
SQ = 256
SH = 128
D_MODEL = 1024
HG = 8
DH = 128
SKV = 4096
SCALE = 0.08838834764831843
NEG = -1e9
W = 384
G = 128


def kernel(x, Wq, K_ext, V_ext, Wo):
    x_bf = x[0].astype(jnp.bfloat16)
    wq_bf = Wq.astype(jnp.bfloat16)
    wo_bf = Wo.astype(jnp.bfloat16)

    def body(x_ref, wq_ref, k_hbm, v_hbm, wo_ref, out_ref,
             k_ref, v_ref,
             xbT, xbB, pT, pB, rsT_s, rsT_r, rsB_s, rsB_r,
             dsem, agT_ss, agT_rs, agB_ss, agB_rs,
             rsT_ss, rsT_rs, rsB_ss, rsB_rs):
        my_i = lax.axis_index("i")
        left = (my_i + N_DEV - 1) % N_DEV
        right = (my_i + 1) % N_DEV

        kv_dmas = []
        for h in range(HG):
            kv_dmas.append(pltpu.make_async_copy(
                k_hbm.at[0, :, my_i * HG + h, :], k_ref.at[h], dsem.at[0, h]))
            kv_dmas.append(pltpu.make_async_copy(
                v_hbm.at[0, :, my_i * HG + h, :], v_ref.at[h], dsem.at[1, h]))
        for c in kv_dmas:
            c.start()

        bsem = pltpu.get_barrier_semaphore()
        for nbr in (left, right):
            pl.semaphore_signal(bsem, inc=1, device_id=(nbr,),
                                device_id_type=pl.DeviceIdType.MESH)
        pl.semaphore_wait(bsem, 2)

        def ag_T(h):
            src = x_ref.at[pl.ds(0, SH)] if h == 0 else xbT.at[h - 1]
            return pltpu.make_async_remote_copy(
                src_ref=src, dst_ref=xbT.at[h],
                send_sem=agT_ss.at[h], recv_sem=agT_rs.at[h],
                device_id=(right,), device_id_type=pl.DeviceIdType.MESH)

        def ag_B(h):
            src = x_ref.at[pl.ds(SH, SH)] if h == 0 else xbB.at[h - 1]
            return pltpu.make_async_remote_copy(
                src_ref=src, dst_ref=xbB.at[h],
                send_sem=agB_ss.at[h], recv_sem=agB_rs.at[h],
                device_id=(left,), device_id_type=pl.DeviceIdType.MESH)

        def rs_T(t):
            return pltpu.make_async_remote_copy(
                src_ref=rsT_s.at[t], dst_ref=rsT_r.at[t],
                send_sem=rsT_ss.at[t], recv_sem=rsT_rs.at[t],
                device_id=(right,), device_id_type=pl.DeviceIdType.MESH)

        def rs_B(t):
            return pltpu.make_async_remote_copy(
                src_ref=rsB_s.at[t], dst_ref=rsB_r.at[t],
                send_sem=rsB_ss.at[t], recv_sem=rsB_rs.at[t],
                device_id=(left,), device_id_type=pl.DeviceIdType.MESH)

        def softmax_rows(s, mask):
            s = jnp.where(mask, s, jnp.float32(NEG)) if mask is not None else s
            m = jnp.max(s, axis=1, keepdims=True)
            w = jnp.exp(s - m)
            d = jnp.sum(w, axis=1, keepdims=True)
            return (w / d).astype(jnp.bfloat16)

        def compute_half(dst, slot, q, top):
            if top:
                xq = x_ref[0:SH] if slot == 0 else xbT[slot - 1]
            else:
                xq = x_ref[SH:SQ] if slot == 0 else xbB[slot - 1]
            qm = jnp.dot(xq, wq_ref[:], preferred_element_type=jnp.float32)
            qm = (qm * SCALE).astype(jnp.bfloat16)
            base = q * SQ + (0 if top else SH)
            start = pl.multiple_of(jnp.maximum(base - SH, 0), 128)
            qi = base + lax.broadcasted_iota(jnp.int32, (SH, W), 0)
            kiw = start + lax.broadcasted_iota(jnp.int32, (SH, W), 1)
            mask_w = (jnp.abs(qi - kiw) <= 128) | (kiw < 32)
            kig = lax.broadcasted_iota(jnp.int32, (SH, G), 1)
            mask_g = (kig < 32) & (start > 0)
            mask = jnp.concatenate([mask_w, mask_g], axis=1)
            parts = []
            for h in range(HG):
                qh = qm[:, h * DH:(h + 1) * DH]
                kcat = jnp.concatenate(
                    [k_ref[h, pl.ds(start, W), :], k_ref[h, :G, :]],
                    axis=0).astype(jnp.bfloat16)
                vcat = jnp.concatenate(
                    [v_ref[h, pl.ds(start, W), :], v_ref[h, :G, :]],
                    axis=0).astype(jnp.bfloat16)
                s = lax.dot_general(qh, kcat, (((1,), (1,)), ((), ())),
                                    preferred_element_type=jnp.float32)
                w = softmax_rows(s, mask)
                parts.append(jnp.dot(w, vcat,
                                     preferred_element_type=jnp.float32))
            ctx = jnp.concatenate(parts, axis=1).astype(jnp.bfloat16)
            dst[slot] = jnp.dot(ctx, wo_ref[:],
                                preferred_element_type=jnp.float32)

            if top:
                @pl.when(q == 0)
                def _():
                    parts32 = []
                    for h in range(HG):
                        q32 = qm[0:32, h * DH:(h + 1) * DH].astype(jnp.float32)
                        s32 = lax.dot_general(
                            q32, k_ref[h], (((1,), (1,)), ((), ())),
                            preferred_element_type=jnp.float32)
                        w32 = softmax_rows(s32, None).astype(jnp.float32)
                        parts32.append(jnp.dot(
                            w32, v_ref[h], preferred_element_type=jnp.float32))
                    ctx32 = jnp.concatenate(
                        parts32, axis=1).astype(jnp.bfloat16)
                    dst[slot, 0:32, :] = jnp.dot(
                        ctx32, wo_ref[:], preferred_element_type=jnp.float32)

        agT0 = ag_T(0)
        agT0.start()
        agB0 = ag_B(0)
        agB0.start()

        for c in kv_dmas:
            c.wait()

        compute_half(pT, 0, my_i, True)
        compute_half(pB, 0, my_i, False)

        agT0.wait()
        agT1 = ag_T(1)
        agT1.start()
        agB0.wait()
        agB1 = ag_B(1)
        agB1.start()

        compute_half(pT, 1, (my_i + 3) % N_DEV, True)
        rsT_s[0] = pT[1].astype(jnp.bfloat16)
        rsT0 = rs_T(0)
        rsT0.start()
        compute_half(pB, 1, (my_i + 1) % N_DEV, False)
        rsB_s[0] = pB[1].astype(jnp.bfloat16)
        rsB0 = rs_B(0)
        rsB0.start()

        agT1.wait()
        agT2 = ag_T(2)
        agT2.start()
        agB1.wait()
        agB2 = ag_B(2)
        agB2.start()

        compute_half(pT, 2, (my_i + 2) % N_DEV, True)
        rsT0.wait()
        pT[2] += rsT_r[0].astype(jnp.float32)
        rsT_s[1] = pT[2].astype(jnp.bfloat16)
        rsT1 = rs_T(1)
        rsT1.start()
        compute_half(pB, 2, (my_i + 2) % N_DEV, False)
        rsB0.wait()
        pB[2] += rsB_r[0].astype(jnp.float32)
        rsB_s[1] = pB[2].astype(jnp.bfloat16)
        rsB1 = rs_B(1)
        rsB1.start()

        agT2.wait()
        agB2.wait()

        compute_half(pT, 3, (my_i + 1) % N_DEV, True)
        rsT1.wait()
        pT[3] += rsT_r[1].astype(jnp.float32)
        rsT_s[2] = pT[3].astype(jnp.bfloat16)
        rsT2 = rs_T(2)
        rsT2.start()
        compute_half(pB, 3, (my_i + 3) % N_DEV, False)
        rsB1.wait()
        pB[3] += rsB_r[1].astype(jnp.float32)
        rsB_s[2] = pB[3].astype(jnp.bfloat16)
        rsB2 = rs_B(2)
        rsB2.start()

        rsT2.wait()
        out_ref[0:SH, :] = pT[0] + rsT_r[2].astype(jnp.float32)
        rsB2.wait()
        out_ref[SH:SQ, :] = pB[0] + rsB_r[2].astype(jnp.float32)

        @functools.partial(pl.run_scoped, sem2=pltpu.SemaphoreType.REGULAR)
        def _(sem2):
            for nbr in (left, right):
                pl.semaphore_signal(sem2, inc=1, device_id=(nbr,),
                                    device_id_type=pl.DeviceIdType.MESH)
            pl.semaphore_wait(sem2, 2)

    out = pl.pallas_call(
        body,
        out_shape=jax.ShapeDtypeStruct((SQ, D_MODEL), jnp.float32),
        in_specs=[
            pl.BlockSpec(memory_space=pltpu.VMEM),
            pl.BlockSpec(memory_space=pltpu.VMEM),
            pl.BlockSpec(memory_space=pl.ANY),
            pl.BlockSpec(memory_space=pl.ANY),
            pl.BlockSpec(memory_space=pltpu.VMEM),
        ],
        out_specs=pl.BlockSpec(memory_space=pltpu.VMEM),
        scratch_shapes=[
            pltpu.VMEM((HG, SKV, DH), jnp.float32),
            pltpu.VMEM((HG, SKV, DH), jnp.float32),
            pltpu.VMEM((3, SH, D_MODEL), jnp.bfloat16),
            pltpu.VMEM((3, SH, D_MODEL), jnp.bfloat16),
            pltpu.VMEM((N_DEV, SH, D_MODEL), jnp.float32),
            pltpu.VMEM((N_DEV, SH, D_MODEL), jnp.float32),
            pltpu.VMEM((3, SH, D_MODEL), jnp.bfloat16),
            pltpu.VMEM((3, SH, D_MODEL), jnp.bfloat16),
            pltpu.VMEM((3, SH, D_MODEL), jnp.bfloat16),
            pltpu.VMEM((3, SH, D_MODEL), jnp.bfloat16),
            pltpu.SemaphoreType.DMA((2, HG)),
            pltpu.SemaphoreType.DMA((3,)),
            pltpu.SemaphoreType.DMA((3,)),
            pltpu.SemaphoreType.DMA((3,)),
            pltpu.SemaphoreType.DMA((3,)),
            pltpu.SemaphoreType.DMA((3,)),
            pltpu.SemaphoreType.DMA((3,)),
            pltpu.SemaphoreType.DMA((3,)),
            pltpu.SemaphoreType.DMA((3,)),
        ],
        compiler_params=pltpu.CompilerParams(
            collective_id=0, vmem_limit_bytes=60 * 1024 * 1024),
    )(x_bf, wq_bf, K_ext, V_ext, wo_bf)
    return out[None]


# device time: 61242 ns/iter; 1.2417x vs baseline; 1.2329x over previous
import functools

import jax
import jax.numpy as jnp
from jax import lax
from jax.experimental import pallas as pl
from jax.experimental.pallas import tpu as pltpu

N_DEV = 4
SQ = 256
SH = 128
D_MODEL = 1024
HG = 8
DH = 128
SKV = 4096
SCALE = 0.08838834764831843
NEG = -1e9
W = 384
G = 128


def kernel(x, Wq, K_ext, V_ext, Wo):
    x_bf = x[0].astype(jnp.bfloat16)
    wq_bf = (Wq * SCALE).astype(jnp.bfloat16)
    wo_bf = Wo.astype(jnp.bfloat16)

    def body(x_ref, wq_ref, k_hbm, v_hbm, wo_ref, out_ref,
             k_ref, v_ref,
             xbT, xbB, pT, pB, rsT_s, rsT_r, rsB_s, rsB_r,
             dsem, agT_ss, agT_rs, agB_ss, agB_rs,
             rsT_ss, rsT_rs, rsB_ss, rsB_rs):
        my_i = lax.axis_index("i")
        left = (my_i + N_DEV - 1) % N_DEV
        right = (my_i + 1) % N_DEV

        kv_dmas = []
        for h in range(HG):
            kv_dmas.append(pltpu.make_async_copy(
                k_hbm.at[0, :, my_i * HG + h, :], k_ref.at[h], dsem.at[0, h]))
            kv_dmas.append(pltpu.make_async_copy(
                v_hbm.at[0, :, my_i * HG + h, :], v_ref.at[h], dsem.at[1, h]))
        for c in kv_dmas:
            c.start()

        bsem = pltpu.get_barrier_semaphore()
        for nbr in (left, right):
            pl.semaphore_signal(bsem, inc=1, device_id=(nbr,),
                                device_id_type=pl.DeviceIdType.MESH)
        pl.semaphore_wait(bsem, 2)

        def ag_T(h):
            src = x_ref.at[pl.ds(0, SH)] if h == 0 else xbT.at[h - 1]
            return pltpu.make_async_remote_copy(
                src_ref=src, dst_ref=xbT.at[h],
                send_sem=agT_ss.at[h], recv_sem=agT_rs.at[h],
                device_id=(right,), device_id_type=pl.DeviceIdType.MESH)

        def ag_B(h):
            src = x_ref.at[pl.ds(SH, SH)] if h == 0 else xbB.at[h - 1]
            return pltpu.make_async_remote_copy(
                src_ref=src, dst_ref=xbB.at[h],
                send_sem=agB_ss.at[h], recv_sem=agB_rs.at[h],
                device_id=(left,), device_id_type=pl.DeviceIdType.MESH)

        def rs_T(t):
            return pltpu.make_async_remote_copy(
                src_ref=rsT_s.at[t], dst_ref=rsT_r.at[t],
                send_sem=rsT_ss.at[t], recv_sem=rsT_rs.at[t],
                device_id=(right,), device_id_type=pl.DeviceIdType.MESH)

        def rs_B(t):
            return pltpu.make_async_remote_copy(
                src_ref=rsB_s.at[t], dst_ref=rsB_r.at[t],
                send_sem=rsB_ss.at[t], recv_sem=rsB_rs.at[t],
                device_id=(left,), device_id_type=pl.DeviceIdType.MESH)

        def softmax_rows(s, mask):
            s = jnp.where(mask, s, jnp.float32(NEG)) if mask is not None else s
            w = jnp.exp(s)
            dinv = 1.0 / jnp.sum(w, axis=1, keepdims=True)
            return w.astype(jnp.bfloat16), dinv

        def compute_half(dst, slot, q, top):
            if top:
                xq = x_ref[0:SH] if slot == 0 else xbT[slot - 1]
            else:
                xq = x_ref[SH:SQ] if slot == 0 else xbB[slot - 1]
            qm = jnp.dot(xq, wq_ref[:],
                         preferred_element_type=jnp.float32)
            qm = qm.astype(jnp.bfloat16)
            base = q * SQ + (0 if top else SH)
            start = pl.multiple_of(jnp.maximum(base - SH, 0), 128)
            qi = base + lax.broadcasted_iota(jnp.int32, (SH, W), 0)
            kiw = start + lax.broadcasted_iota(jnp.int32, (SH, W), 1)
            mask_w = (jnp.abs(qi - kiw) <= 128) | (kiw < 32)
            kig = lax.broadcasted_iota(jnp.int32, (SH, G), 1)
            mask_g = (kig < 32) & (start > 0)
            mask = jnp.concatenate([mask_w, mask_g], axis=1)
            parts = []
            for h in range(HG):
                qh = qm[:, h * DH:(h + 1) * DH]
                kcat = jnp.concatenate(
                    [k_ref[h, pl.ds(start, W), :], k_ref[h, :G, :]],
                    axis=0).astype(jnp.bfloat16)
                vcat = jnp.concatenate(
                    [v_ref[h, pl.ds(start, W), :], v_ref[h, :G, :]],
                    axis=0).astype(jnp.bfloat16)
                s = lax.dot_general(qh, kcat, (((1,), (1,)), ((), ())),
                                    preferred_element_type=jnp.float32)
                w, dinv = softmax_rows(s, mask)
                parts.append(jnp.dot(
                    w, vcat, preferred_element_type=jnp.float32) * dinv)
            ctx = jnp.concatenate(parts, axis=1).astype(jnp.bfloat16)
            dst[slot] = jnp.dot(ctx, wo_ref[:],
                                preferred_element_type=jnp.float32)

            if top:
                @pl.when(q == 0)
                def _():
                    parts32 = []
                    for h in range(HG):
                        q32 = qm[0:32, h * DH:(h + 1) * DH].astype(jnp.float32)
                        s32 = lax.dot_general(
                            q32, k_ref[h], (((1,), (1,)), ((), ())),
                            preferred_element_type=jnp.float32)
                        w32, dinv32 = softmax_rows(s32, None)
                        parts32.append(jnp.dot(
                            w32.astype(jnp.float32), v_ref[h],
                            preferred_element_type=jnp.float32) * dinv32)
                    ctx32 = jnp.concatenate(
                        parts32, axis=1).astype(jnp.bfloat16)
                    dst[slot, 0:32, :] = jnp.dot(
                        ctx32, wo_ref[:], preferred_element_type=jnp.float32)

        agT0 = ag_T(0)
        agT0.start()
        agB0 = ag_B(0)
        agB0.start()

        for c in kv_dmas:
            c.wait()

        compute_half(pT, 0, my_i, True)
        compute_half(pB, 0, my_i, False)

        agT0.wait()
        agT1 = ag_T(1)
        agT1.start()
        agB0.wait()
        agB1 = ag_B(1)
        agB1.start()

        compute_half(pT, 1, (my_i + 3) % N_DEV, True)
        rsT_s[0] = pT[1].astype(jnp.bfloat16)
        rsT0 = rs_T(0)
        rsT0.start()
        compute_half(pB, 1, (my_i + 1) % N_DEV, False)
        rsB_s[0] = pB[1].astype(jnp.bfloat16)
        rsB0 = rs_B(0)
        rsB0.start()

        agT1.wait()
        agT2 = ag_T(2)
        agT2.start()
        agB1.wait()
        agB2 = ag_B(2)
        agB2.start()

        compute_half(pT, 2, (my_i + 2) % N_DEV, True)
        rsT0.wait()
        pT[2] += rsT_r[0].astype(jnp.float32)
        rsT_s[1] = pT[2].astype(jnp.bfloat16)
        rsT1 = rs_T(1)
        rsT1.start()
        compute_half(pB, 2, (my_i + 2) % N_DEV, False)
        rsB0.wait()
        pB[2] += rsB_r[0].astype(jnp.float32)
        rsB_s[1] = pB[2].astype(jnp.bfloat16)
        rsB1 = rs_B(1)
        rsB1.start()

        agT2.wait()
        agB2.wait()

        compute_half(pT, 3, (my_i + 1) % N_DEV, True)
        rsT1.wait()
        pT[3] += rsT_r[1].astype(jnp.float32)
        rsT_s[2] = pT[3].astype(jnp.bfloat16)
        rsT2 = rs_T(2)
        rsT2.start()
        compute_half(pB, 3, (my_i + 3) % N_DEV, False)
        rsB1.wait()
        pB[3] += rsB_r[1].astype(jnp.float32)
        rsB_s[2] = pB[3].astype(jnp.bfloat16)
        rsB2 = rs_B(2)
        rsB2.start()

        rsT2.wait()
        out_ref[0:SH, :] = pT[0] + rsT_r[2].astype(jnp.float32)
        rsB2.wait()
        out_ref[SH:SQ, :] = pB[0] + rsB_r[2].astype(jnp.float32)

        @functools.partial(pl.run_scoped, sem2=pltpu.SemaphoreType.REGULAR)
        def _(sem2):
            for nbr in (left, right):
                pl.semaphore_signal(sem2, inc=1, device_id=(nbr,),
                                    device_id_type=pl.DeviceIdType.MESH)
            pl.semaphore_wait(sem2, 2)

    out = pl.pallas_call(
        body,
        out_shape=jax.ShapeDtypeStruct((SQ, D_MODEL), jnp.float32),
        in_specs=[
            pl.BlockSpec(memory_space=pltpu.VMEM),
            pl.BlockSpec(memory_space=pltpu.VMEM),
            pl.BlockSpec(memory_space=pl.ANY),
            pl.BlockSpec(memory_space=pl.ANY),
            pl.BlockSpec(memory_space=pltpu.VMEM),
        ],
        out_specs=pl.BlockSpec(memory_space=pltpu.VMEM),
        scratch_shapes=[
            pltpu.VMEM((HG, SKV, DH), jnp.float32),
            pltpu.VMEM((HG, SKV, DH), jnp.float32),
            pltpu.VMEM((3, SH, D_MODEL), jnp.bfloat16),
            pltpu.VMEM((3, SH, D_MODEL), jnp.bfloat16),
            pltpu.VMEM((N_DEV, SH, D_MODEL), jnp.float32),
            pltpu.VMEM((N_DEV, SH, D_MODEL), jnp.float32),
            pltpu.VMEM((3, SH, D_MODEL), jnp.bfloat16),
            pltpu.VMEM((3, SH, D_MODEL), jnp.bfloat16),
            pltpu.VMEM((3, SH, D_MODEL), jnp.bfloat16),
            pltpu.VMEM((3, SH, D_MODEL), jnp.bfloat16),
            pltpu.SemaphoreType.DMA((2, HG)),
            pltpu.SemaphoreType.DMA((3,)),
            pltpu.SemaphoreType.DMA((3,)),
            pltpu.SemaphoreType.DMA((3,)),
            pltpu.SemaphoreType.DMA((3,)),
            pltpu.SemaphoreType.DMA((3,)),
            pltpu.SemaphoreType.DMA((3,)),
            pltpu.SemaphoreType.DMA((3,)),
            pltpu.SemaphoreType.DMA((3,)),
        ],
        compiler_params=pltpu.CompilerParams(
            collective_id=0, vmem_limit_bytes=60 * 1024 * 1024),
    )(x_bf, wq_bf, K_ext, V_ext, wo_bf)
    return out[None]
